# Initial kernel scaffold; baseline (speedup 1.0000x reference)
import jax
import jax.numpy as jnp
from jax.experimental import pallas as pl


def kernel(input_ids, table):
    raise NotImplementedError("write your pallas kernel here")


# speedup vs baseline: 2.4882x; 2.4882x over previous
import functools

import jax
import jax.numpy as jnp
from jax import lax
from jax.experimental import pallas as pl
from jax.experimental.pallas import tpu as pltpu
from jax.experimental.pallas import tpu_sc as plsc

_D = 16
_SLAB = 2048
_SUB = 128


@functools.cache
def _make_gather(total: int):
    info = plsc.get_sparse_core_info()
    nw = info.num_cores * info.num_subcores
    per_w = total // nw
    assert per_w * nw == total and per_w % _SLAB == 0
    n_slab = per_w // _SLAB
    mesh = plsc.VectorSubcoreMesh(core_axis_name="c", subcore_axis_name="s")

    @functools.partial(
        pl.kernel,
        mesh=mesh,
        compiler_params=pltpu.CompilerParams(use_tc_tiling_on_sc=False),
        out_type=jax.ShapeDtypeStruct((total, _D), jnp.float32),
        scratch_types=[
            pltpu.VMEM((_SLAB,), jnp.int32),
            pltpu.VMEM((_SLAB, _D), jnp.float32),
            pltpu.SemaphoreType.DMA,
        ],
    )
    def gather_kernel(ids_hbm, table_hbm, out_hbm, idx_v, rows_v, sem):
        wid = lax.axis_index("s") * info.num_cores + lax.axis_index("c")
        base = wid * per_w

        def body(i, carry):
            off = base + i * _SLAB
            pltpu.sync_copy(ids_hbm.at[pl.ds(off, _SLAB)], idx_v)
            copies = [
                pltpu.async_copy(
                    table_hbm.at[idx_v.at[pl.ds(j * _SUB, _SUB)]],
                    rows_v.at[pl.ds(j * _SUB, _SUB)],
                    sem,
                )
                for j in range(_SLAB // _SUB)
            ]
            for c in copies:
                c.wait()
            pltpu.sync_copy(rows_v, out_hbm.at[pl.ds(off, _SLAB)])
            return carry

        lax.fori_loop(0, n_slab, body, 0)

    return gather_kernel


def kernel(input_ids, table):
    b, h = input_ids.shape
    flat = input_ids.reshape(b * h).astype(jnp.int32)
    out = _make_gather(b * h)(flat, table)
    return out.reshape(b, h, _D)


# speedup vs baseline: 2.5559x; 1.0272x over previous
import functools

import jax
import jax.numpy as jnp
from jax import lax
from jax.experimental import pallas as pl
from jax.experimental.pallas import tpu as pltpu
from jax.experimental.pallas import tpu_sc as plsc

_D = 16
_SLAB = 2048
_SUB = 128
_NBUF = 2


@functools.cache
def _make_gather(total: int):
    info = plsc.get_sparse_core_info()
    nw = info.num_cores * info.num_subcores
    per_w = total // nw
    assert per_w * nw == total and per_w % (_SLAB * _NBUF) == 0
    n_slab = per_w // _SLAB
    n_outer = n_slab // _NBUF
    rows_bytes = _SLAB * _D * 4
    mesh = plsc.VectorSubcoreMesh(core_axis_name="c", subcore_axis_name="s")

    @functools.partial(
        pl.kernel,
        mesh=mesh,
        compiler_params=pltpu.CompilerParams(use_tc_tiling_on_sc=False),
        out_type=jax.ShapeDtypeStruct((total, _D), jnp.float32),
        scratch_types=[
            pltpu.VMEM((_NBUF, _SLAB), jnp.int32),
            pltpu.VMEM((_NBUF, _SLAB, _D), jnp.float32),
            pltpu.SemaphoreType.DMA((_NBUF,)),
            pltpu.SemaphoreType.DMA((_NBUF,)),
            pltpu.SemaphoreType.DMA((_NBUF,)),
        ],
    )
    def gather_kernel(ids_hbm, table_hbm, out_hbm, idx_v, rows_v,
                      sem_i, sem_g, sem_s):
        wid = lax.axis_index("s") * info.num_cores + lax.axis_index("c")
        base = wid * per_w

        def start_idx_load(slab, b):
            pltpu.async_copy(
                ids_hbm.at[pl.ds(base + slab * _SLAB, _SLAB)],
                idx_v.at[b], sem_i.at[b])

        def fire_gathers(b):
            for j in range(_SLAB // _SUB):
                pltpu.async_copy(
                    table_hbm.at[idx_v.at[b, pl.ds(j * _SUB, _SUB)]],
                    rows_v.at[b, pl.ds(j * _SUB, _SUB)],
                    sem_g.at[b])

        def wait_gathers(b):
            for j in range(_SLAB // _SUB):
                pltpu.make_async_copy(
                    table_hbm.at[idx_v.at[b, pl.ds(j * _SUB, _SUB)]],
                    rows_v.at[b, pl.ds(j * _SUB, _SUB)],
                    sem_g.at[b]).wait()

        def start_store(slab, b):
            pltpu.async_copy(
                rows_v.at[b],
                out_hbm.at[pl.ds(base + slab * _SLAB, _SLAB)],
                sem_s.at[b])

        def wait_store(b):
            pltpu.make_async_copy(
                rows_v.at[b],
                out_hbm.at[pl.ds(base, _SLAB)],
                sem_s.at[b]).wait()

        def wait_idx(b):
            pltpu.make_async_copy(
                ids_hbm.at[pl.ds(base, _SLAB)],
                idx_v.at[b], sem_i.at[b]).wait()

        for b in range(_NBUF):
            start_idx_load(b, b)

        def body(k, carry):
            i0 = k * _NBUF

            @pl.when(k > 0)
            def _():
                for b in range(_NBUF):
                    wait_store(b)

            for b in range(_NBUF):
                wait_idx(b)
                fire_gathers(b)
            for b in range(_NBUF):
                wait_gathers(b)
                start_store(i0 + b, b)

                @pl.when(i0 + b + _NBUF < n_slab)
                def _():
                    start_idx_load(i0 + b + _NBUF, b)

            return carry

        lax.fori_loop(0, n_outer, body, 0)

        for b in range(_NBUF):
            wait_store(b)

    return gather_kernel


def kernel(input_ids, table):
    b, h = input_ids.shape
    flat = input_ids.reshape(b * h).astype(jnp.int32)
    out = _make_gather(b * h)(flat, table)
    return out.reshape(b, h, _D)
